# baseline (speedup 1.0000x reference)
import functools

import jax
import jax.numpy as jnp
from jax.experimental import pallas as pl

N = 128
R = 7
O = 32
NR = N * R
RO = R * O


def _gcn_kernel(x_ref, wcat_ref, rmask_ref, sel_ref, s7_ref, bias_ref, out_ref):
    xb = x_ref[0]
    m = jax.lax.dot_general(
        xb, wcat_ref[...], (((1,), (0,)), ((), ())),
        preferred_element_type=jnp.float32)
    ones = jnp.ones((N, 1), dtype=jnp.float32)
    degc = jax.lax.dot_general(
        xb, ones, (((0,), (0,)), ((), ())),
        preferred_element_type=jnp.float32)
    recip = 1.0 / jnp.maximum(degc, 1.0)
    t = jax.lax.dot_general(
        xb, m[:, :RO], (((0,), (0,)), ((), ())),
        preferred_element_type=jnp.float32)
    masked = t * rmask_ref[...] * recip
    u = jax.lax.dot_general(
        masked, s7_ref[...], (((1,), (0,)), ((), ())),
        preferred_element_type=jnp.float32)
    out_rel = jax.lax.dot_general(
        sel_ref[...], u, (((1,), (0,)), ((), ())),
        preferred_element_type=jnp.float32)
    out_ref[0] = out_rel + m[:, RO:] + bias_ref[...]


@jax.jit
def kernel(x, W_rel, W_root, bias):
    B = x.shape[0]
    x2 = x.reshape(B, N, NR)
    w_big = jnp.einsum('rfo,rs->frso', W_rel, jnp.eye(R, dtype=x.dtype))
    w_big = w_big.reshape(NR, RO)
    gw = jnp.broadcast_to(W_root[:, None, :] / R, (N, R, O)).reshape(NR, O)
    w_cat = jnp.concatenate([w_big, gw], axis=1)
    row_r = jnp.arange(NR, dtype=jnp.int32) % R
    col_r = jnp.arange(RO, dtype=jnp.int32) // O
    rmask = (row_r[:, None] == col_r[None, :]).astype(x.dtype)
    s7 = jnp.tile(jnp.eye(O, dtype=x.dtype), (R, 1))
    sel = jnp.repeat(jnp.eye(N, dtype=x.dtype), R, axis=1)
    bias2 = bias.reshape(1, O)

    grid = (B,)
    return pl.pallas_call(
        _gcn_kernel,
        grid=grid,
        in_specs=[
            pl.BlockSpec((1, N, NR), lambda b: (b, 0, 0)),
            pl.BlockSpec((NR, RO + O), lambda b: (0, 0)),
            pl.BlockSpec((NR, RO), lambda b: (0, 0)),
            pl.BlockSpec((N, NR), lambda b: (0, 0)),
            pl.BlockSpec((RO, O), lambda b: (0, 0)),
            pl.BlockSpec((1, O), lambda b: (0, 0)),
        ],
        out_specs=pl.BlockSpec((1, N, O), lambda b: (b, 0, 0)),
        out_shape=jax.ShapeDtypeStruct((B, N, O), x.dtype),
    )(x2, w_cat, rmask, sel, s7, bias2)


# speedup vs baseline: 2.3906x; 2.3906x over previous
import jax
import jax.numpy as jnp
from jax.experimental import pallas as pl

N = 128
R = 7
O = 32


def _gcn_kernel(x_ref, wrel_ref, wroot_ref, bias_ref, out_ref):
    blk = x_ref[0]
    deg = jnp.sum(blk, axis=1, keepdims=True)
    recip = 1.0 / jnp.maximum(deg, 1.0)
    m = jax.lax.dot_general(
        blk, wrel_ref[...], (((2,), (1,)), ((0,), (0,))),
        preferred_element_type=jnp.float32)
    at = (blk * recip).reshape(R * N, N)
    out_rel = jax.lax.dot_general(
        at, m.reshape(R * N, O), (((0,), (0,)), ((), ())),
        preferred_element_type=jnp.float32)
    hroot = jnp.mean(blk, axis=0)
    root = jax.lax.dot_general(
        hroot, wroot_ref[...], (((1,), (0,)), ((), ())),
        preferred_element_type=jnp.float32)
    out_ref[0] = out_rel + root + bias_ref[...]


@jax.jit
def kernel(x, W_rel, W_root, bias):
    B = x.shape[0]
    xt = jnp.transpose(x, (0, 3, 1, 2))
    bias2 = bias.reshape(1, O)
    return pl.pallas_call(
        _gcn_kernel,
        grid=(B,),
        in_specs=[
            pl.BlockSpec((1, R, N, N), lambda b: (b, 0, 0, 0)),
            pl.BlockSpec((R, N, O), lambda b: (0, 0, 0)),
            pl.BlockSpec((N, O), lambda b: (0, 0)),
            pl.BlockSpec((1, O), lambda b: (0, 0)),
        ],
        out_specs=pl.BlockSpec((1, N, O), lambda b: (b, 0, 0)),
        out_shape=jax.ShapeDtypeStruct((B, N, O), x.dtype),
    )(xt, W_rel, W_root, bias2)
